# baseline (speedup 1.0000x reference)
import jax
import jax.numpy as jnp
from jax import lax
from jax.experimental import pallas as pl
from jax.experimental.pallas import tpu as pltpu
from jax.experimental.pallas import tpu_sc as plsc

N_ENT = 1000000
BATCH = 16384
RANK = 32
LANES = 128
NUM_CORES = 2
NUM_SUBCORES = 16
NUM_WORKERS = NUM_CORES * NUM_SUBCORES
B_PER_W = BATCH // NUM_WORKERS
NBUF = 16
GROUP = 16
N_GROUPS = B_PER_W // GROUP


def _gather_body(head_hbm, table_hbm, out_hbm, idx_v, ring_v, out_v, sems):
    wid = lax.axis_index("s") * NUM_CORES + lax.axis_index("c")
    base = wid * B_PER_W
    pltpu.sync_copy(head_hbm.at[pl.ds(base, B_PER_W)], idx_v.at[pl.ds(0, B_PER_W)])

    iota = lax.iota(jnp.int32, 16)
    rvec = iota >> 3
    svec = iota & 7

    def fetch(i, slot):
        col = pl.multiple_of((i >> 7) * LANES, LANES)
        pltpu.make_async_copy(
            table_hbm.at[:, :, pl.ds(col, LANES)],
            ring_v.at[slot],
            sems.at[slot],
        ).start()

    def wait(slot):
        pltpu.make_async_copy(
            table_hbm.at[:, :, pl.ds(0, LANES)],
            ring_v.at[slot],
            sems.at[slot],
        ).wait()

    v0 = idx_v[pl.ds(0, GROUP)]
    for k in range(NBUF):
        fetch(v0[k], k)

    def step(g, _):
        qbase = g * GROUP
        vg = idx_v[pl.ds(qbase, GROUP)]
        vnext = idx_v[pl.ds(qbase + GROUP, GROUP)]
        for k in range(GROUP):
            slot = k % NBUF
            wait(slot)
            i = vg[k]
            lvec = jnp.full((16,), i & (LANES - 1), jnp.int32)
            qvec = jnp.full((16,), qbase + k, jnp.int32)
            sl = jnp.full((16,), slot, jnp.int32)
            x0 = plsc.load_gather(ring_v, [sl, rvec, svec, lvec])
            x1 = plsc.load_gather(ring_v, [sl, rvec + 2, svec, lvec])
            plsc.store_scatter(out_v, [rvec, svec, qvec], x0)
            plsc.store_scatter(out_v, [rvec + 2, svec, qvec], x1)
            if k < GROUP - NBUF:
                fetch(vg[k + NBUF], slot)
            else:

                @pl.when(g + 1 < N_GROUPS)
                def _prefetch():
                    fetch(vnext[k - (GROUP - NBUF)], slot)

        return 0

    lax.fori_loop(0, N_GROUPS, step, 0)
    pltpu.sync_copy(out_v, out_hbm.at[:, :, pl.ds(base, B_PER_W)])


@jax.jit
def _gather(head_idx, table_v):
    k = pl.kernel(
        _gather_body,
        out_type=jax.ShapeDtypeStruct((4, 8, BATCH), jnp.float32),
        mesh=plsc.VectorSubcoreMesh(core_axis_name="c", subcore_axis_name="s"),
        scratch_types=[
            pltpu.VMEM((B_PER_W + GROUP,), jnp.int32),
            pltpu.VMEM((NBUF, 4, 8, LANES), jnp.float32),
            pltpu.VMEM((4, 8, B_PER_W), jnp.float32),
            pltpu.SemaphoreType.DMA((NBUF,)),
        ],
        compiler_params=pltpu.CompilerParams(needs_layout_passes=False),
    )
    return k(head_idx, table_v)


def kernel(head, entity_weight, rel_weight, bh_weight, bt_weight):
    head_idx = head.astype(jnp.int32)
    table_v = entity_weight.T.reshape(4, 8, N_ENT)
    out_v = _gather(head_idx, table_v)
    return out_v.transpose(2, 0, 1).reshape(BATCH, RANK)


# speedup vs baseline: 1.0293x; 1.0293x over previous
import jax
import jax.numpy as jnp
from jax import lax
from jax.experimental import pallas as pl
from jax.experimental.pallas import tpu as pltpu
from jax.experimental.pallas import tpu_sc as plsc

N_ENT = 1000000
BATCH = 16384
RANK = 32
LANES = 128
NUM_CORES = 2
NUM_SUBCORES = 16
NUM_WORKERS = NUM_CORES * NUM_SUBCORES
B_PER_W = BATCH // NUM_WORKERS
NBUF = 8
GROUP = 16
N_GROUPS = B_PER_W // GROUP


def _gather_body(head_hbm, table_hbm, out_hbm, idx_v, ring_v, out_v, sems):
    wid = lax.axis_index("s") * NUM_CORES + lax.axis_index("c")
    base = wid * B_PER_W
    pltpu.sync_copy(head_hbm.at[pl.ds(base, B_PER_W)], idx_v.at[pl.ds(0, B_PER_W)])

    iota = lax.iota(jnp.int32, 16)
    rvec = iota >> 3
    svec = iota & 7

    def fetch(i, slot):
        col = pl.multiple_of((i >> 7) * LANES, LANES)
        for r in range(4):
            pltpu.make_async_copy(
                table_hbm.at[r, :, pl.ds(col, LANES)],
                ring_v.at[slot, r],
                sems.at[slot],
            ).start()

    def wait(slot):
        pltpu.make_async_copy(
            table_hbm.at[:, :, pl.ds(0, LANES)],
            ring_v.at[slot],
            sems.at[slot],
        ).wait()

    v0 = idx_v[pl.ds(0, GROUP)]
    for k in range(NBUF):
        fetch(v0[k], k)

    def step(g, _):
        qbase = g * GROUP
        vg = idx_v[pl.ds(qbase, GROUP)]
        vnext = idx_v[pl.ds(qbase + GROUP, GROUP)]
        for k in range(GROUP):
            slot = k % NBUF
            wait(slot)
            i = vg[k]
            lvec = jnp.full((16,), i & (LANES - 1), jnp.int32)
            qvec = jnp.full((16,), qbase + k, jnp.int32)
            sl = jnp.full((16,), slot, jnp.int32)
            x0 = plsc.load_gather(ring_v, [sl, rvec, svec, lvec])
            x1 = plsc.load_gather(ring_v, [sl, rvec + 2, svec, lvec])
            plsc.store_scatter(out_v, [rvec, svec, qvec], x0)
            plsc.store_scatter(out_v, [rvec + 2, svec, qvec], x1)
            if k < GROUP - NBUF:
                fetch(vg[k + NBUF], slot)
            else:

                @pl.when(g + 1 < N_GROUPS)
                def _prefetch():
                    fetch(vnext[k - (GROUP - NBUF)], slot)

        return 0

    lax.fori_loop(0, N_GROUPS, step, 0)
    pltpu.sync_copy(out_v, out_hbm.at[:, :, pl.ds(base, B_PER_W)])


@jax.jit
def _gather(head_idx, table_v):
    k = pl.kernel(
        _gather_body,
        out_type=jax.ShapeDtypeStruct((4, 8, BATCH), jnp.float32),
        mesh=plsc.VectorSubcoreMesh(core_axis_name="c", subcore_axis_name="s"),
        scratch_types=[
            pltpu.VMEM((B_PER_W + GROUP,), jnp.int32),
            pltpu.VMEM((NBUF, 4, 8, LANES), jnp.float32),
            pltpu.VMEM((4, 8, B_PER_W), jnp.float32),
            pltpu.SemaphoreType.DMA((NBUF,)),
        ],
        compiler_params=pltpu.CompilerParams(needs_layout_passes=False),
    )
    return k(head_idx, table_v)


def kernel(head, entity_weight, rel_weight, bh_weight, bt_weight):
    head_idx = head.astype(jnp.int32)
    table_v = entity_weight.T.reshape(4, 8, N_ENT)
    out_v = _gather(head_idx, table_v)
    return out_v.transpose(2, 0, 1).reshape(BATCH, RANK)


# speedup vs baseline: 1.1108x; 1.0792x over previous
import jax
import jax.numpy as jnp
from jax import lax
from jax.experimental import pallas as pl
from jax.experimental.pallas import tpu as pltpu
from jax.experimental.pallas import tpu_sc as plsc

N_ENT = 1000000
BATCH = 16384
RANK = 32
LANES = 128
NUM_CORES = 2
NUM_SUBCORES = 16
NUM_WORKERS = NUM_CORES * NUM_SUBCORES
B_PER_W = BATCH // NUM_WORKERS
NBUF = 8
GROUP = 16
N_GROUPS = B_PER_W // GROUP


def _gather_body(head_hbm, table_hbm, out_hbm, idx_v, ring_v, out_v, sems):
    wid = lax.axis_index("s") * NUM_CORES + lax.axis_index("c")
    base = wid * B_PER_W
    pltpu.sync_copy(head_hbm.at[pl.ds(base, B_PER_W)], idx_v.at[pl.ds(0, B_PER_W)])

    iota = lax.iota(jnp.int32, 16)
    rvec = iota >> 3
    svec = iota & 7

    def fetch(i, slot):
        col = pl.multiple_of((i >> 7) * LANES, LANES)
        small = (i & (LANES - 1)) < (LANES // 2)

        @pl.when(small)
        def _small():
            for r in range(4):
                pltpu.make_async_copy(
                    table_hbm.at[r, :, pl.ds(col, LANES // 2)],
                    ring_v.at[slot, r, :, pl.ds(0, LANES // 2)],
                    sems.at[slot],
                ).start()

        @pl.when(jnp.logical_not(small))
        def _big():
            for r in range(4):
                pltpu.make_async_copy(
                    table_hbm.at[r, :, pl.ds(col, LANES)],
                    ring_v.at[slot, r],
                    sems.at[slot],
                ).start()

    def wait(i, slot):
        small = (i & (LANES - 1)) < (LANES // 2)

        @pl.when(small)
        def _small():
            pltpu.make_async_copy(
                table_hbm.at[:, :, pl.ds(0, LANES // 2)],
                ring_v.at[slot, :, :, pl.ds(0, LANES // 2)],
                sems.at[slot],
            ).wait()

        @pl.when(jnp.logical_not(small))
        def _big():
            pltpu.make_async_copy(
                table_hbm.at[:, :, pl.ds(0, LANES)],
                ring_v.at[slot],
                sems.at[slot],
            ).wait()

    v0 = idx_v[pl.ds(0, GROUP)]
    for k in range(NBUF):
        fetch(v0[k], k)

    def step(g, _):
        qbase = g * GROUP
        vg = idx_v[pl.ds(qbase, GROUP)]
        vnext = idx_v[pl.ds(qbase + GROUP, GROUP)]
        for k in range(GROUP):
            slot = k % NBUF
            i = vg[k]
            wait(i, slot)
            lvec = jnp.full((16,), i & (LANES - 1), jnp.int32)
            qvec = jnp.full((16,), qbase + k, jnp.int32)
            sl = jnp.full((16,), slot, jnp.int32)
            x0 = plsc.load_gather(ring_v, [sl, rvec, svec, lvec])
            x1 = plsc.load_gather(ring_v, [sl, rvec + 2, svec, lvec])
            plsc.store_scatter(out_v, [rvec, svec, qvec], x0)
            plsc.store_scatter(out_v, [rvec + 2, svec, qvec], x1)
            if k < GROUP - NBUF:
                fetch(vg[k + NBUF], slot)
            else:

                @pl.when(g + 1 < N_GROUPS)
                def _prefetch():
                    fetch(vnext[k - (GROUP - NBUF)], slot)

        return 0

    lax.fori_loop(0, N_GROUPS, step, 0)
    pltpu.sync_copy(out_v, out_hbm.at[:, :, pl.ds(base, B_PER_W)])


@jax.jit
def _gather(head_idx, table_v):
    k = pl.kernel(
        _gather_body,
        out_type=jax.ShapeDtypeStruct((4, 8, BATCH), jnp.float32),
        mesh=plsc.VectorSubcoreMesh(core_axis_name="c", subcore_axis_name="s"),
        scratch_types=[
            pltpu.VMEM((B_PER_W + GROUP,), jnp.int32),
            pltpu.VMEM((NBUF, 4, 8, LANES), jnp.float32),
            pltpu.VMEM((4, 8, B_PER_W), jnp.float32),
            pltpu.SemaphoreType.DMA((NBUF,)),
        ],
        compiler_params=pltpu.CompilerParams(needs_layout_passes=False),
    )
    return k(head_idx, table_v)


def kernel(head, entity_weight, rel_weight, bh_weight, bt_weight):
    head_idx = head.astype(jnp.int32)
    table_v = entity_weight.T.reshape(4, 8, N_ENT)
    out_v = _gather(head_idx, table_v)
    return out_v.transpose(2, 0, 1).reshape(BATCH, RANK)
